# baseline (speedup 1.0000x reference)
import functools

import jax
import jax.numpy as jnp
from jax import lax
from jax.experimental import pallas as pl
from jax.experimental.pallas import tpu as pltpu
from jax.experimental.pallas import tpu_sc as plsc

N = 10000
N2 = 10240
E = 320000
DIN = 128
DHID = 256
DOUT = 128
SAMP = 10
SLOT = 10
NW = 32
EPW = E // NW
GPW = EPW // 16
NPW = N2 // NW
NPW_A = 256
NPW_B = 384
NRANGE = 3
RN = 3456
SLAB = SLOT * RN
CH = 8
NCH = NPW // CH

_SC_PARAMS = pltpu.CompilerParams(needs_layout_passes=False)


def _sc_mesh():
    return plsc.VectorSubcoreMesh(core_axis_name="c", subcore_axis_name="s")


def _dup16_body(rowt_ref, dupt_ref):
    x = rowt_ref[...]
    sub = lax.broadcasted_iota(jnp.int32, x.shape, 0)
    dup = jnp.zeros(x.shape, jnp.int32)
    for k in range(1, 16):
        rk = jnp.concatenate([x[16 - k:, :], x[:16 - k, :]], axis=0)
        eq = jnp.logical_and(x == rk, sub >= k)
        dup = dup + jnp.where(eq, 1, 0)
    dupt_ref[...] = dup


def _dup16(rowt):
    g = rowt.shape[1]
    bw = 2048
    grid = (pl.cdiv(g, bw),)
    return pl.pallas_call(
        _dup16_body,
        out_shape=jax.ShapeDtypeStruct((16, g), jnp.int32),
        grid=grid,
        in_specs=[pl.BlockSpec((16, bw), lambda i: (0, i))],
        out_specs=pl.BlockSpec((16, bw), lambda i: (0, i)),
    )(rowt)


def _count_kernel(row_hbm, dup_hbm, hist_hbm, lrank_hbm,
                  rowsv, dupv, cntv, lrankv):
    cid = lax.axis_index("c")
    sid = lax.axis_index("s")
    wid = cid * 16 + sid
    base = wid * EPW
    pltpu.sync_copy(row_hbm.at[pl.ds(base, EPW)], rowsv)
    pltpu.sync_copy(dup_hbm.at[pl.ds(base, EPW)], dupv)
    zeros16 = jnp.zeros((16,), jnp.int32)

    def zbody(i, carry):
        cntv[pl.ds(i * 16, 16)] = zeros16
        return carry

    lax.fori_loop(0, N2 // 16, zbody, jnp.int32(0))
    ones16 = jnp.ones((16,), jnp.int32)

    def body(g, carry):
        r16 = rowsv[pl.ds(g * 16, 16)]
        d16 = dupv[pl.ds(g * 16, 16)]
        c16 = plsc.load_gather(cntv, [r16])
        lrankv[pl.ds(g * 16, 16)] = c16 + d16
        plsc.addupdate_scatter(cntv, [r16], ones16)
        return carry

    lax.fori_loop(0, GPW, body, jnp.int32(0))
    pltpu.sync_copy(cntv, hist_hbm.at[wid])
    pltpu.sync_copy(lrankv, lrank_hbm.at[pl.ds(base, EPW)])


def _count(row, dup):
    k = functools.partial(
        pl.kernel,
        out_type=[jax.ShapeDtypeStruct((NW, N2), jnp.int32),
                  jax.ShapeDtypeStruct((E,), jnp.int32)],
        mesh=_sc_mesh(),
        compiler_params=_SC_PARAMS,
        scratch_types=[pltpu.VMEM((EPW,), jnp.int32),
                       pltpu.VMEM((EPW,), jnp.int32),
                       pltpu.VMEM((N2,), jnp.int32),
                       pltpu.VMEM((EPW,), jnp.int32)],
    )(_count_kernel)
    return k(row, dup)


def _offsets_body(hist_ref, offs_ref, inv_ref):
    h = hist_ref[...]
    acc = jnp.zeros((1, h.shape[1]), jnp.int32)
    rows = []
    for w in range(NW):
        rows.append(acc)
        acc = acc + h[w:w + 1, :]
    offs_ref[...] = jnp.concatenate(rows, axis=0)
    deg = jnp.minimum(acc, SAMP)
    deg = jnp.maximum(deg, 1).astype(jnp.float32)
    inv = 1.0 / deg
    r = lax.broadcasted_iota(jnp.int32, (128, 128), 0)
    c = lax.broadcasted_iota(jnp.int32, (128, 128), 1)
    ident = (r == c).astype(jnp.float32)
    inv_ref[...] = lax.dot_general(
        ident, inv, (((1,), (1,)), ((), ())),
        precision=lax.Precision.HIGHEST,
        preferred_element_type=jnp.float32)


def _offsets(hist):
    grid = (N2 // 128,)
    return pl.pallas_call(
        _offsets_body,
        out_shape=[jax.ShapeDtypeStruct((NW, N2), jnp.int32),
                   jax.ShapeDtypeStruct((N2, 1), jnp.float32)],
        grid=grid,
        in_specs=[pl.BlockSpec((NW, 128), lambda i: (0, i))],
        out_specs=[pl.BlockSpec((NW, 128), lambda i: (0, i)),
                   pl.BlockSpec((128, 1), lambda i: (i, 0))],
    )(hist)


def _nbr_build_kernel(row_hbm, col_hbm, lrank_hbm, offs_hbm, z_hbm,
                      part_hbm, rowsv, colsv, lrankv, offsv, slabv):
    cid = lax.axis_index("c")
    sid = lax.axis_index("s")
    wid = cid * 16 + sid
    base = wid * EPW
    pltpu.sync_copy(row_hbm.at[pl.ds(base, EPW)], rowsv)
    pltpu.sync_copy(col_hbm.at[pl.ds(base, EPW)], colsv)
    pltpu.sync_copy(lrank_hbm.at[pl.ds(base, EPW)], lrankv)
    pltpu.sync_copy(offs_hbm.at[wid], offsv)

    for p in range(NRANGE):
        pltpu.sync_copy(z_hbm, slabv)
        nbase = p * RN

        def body(g, carry):
            r16 = rowsv[pl.ds(g * 16, 16)]
            c16 = colsv[pl.ds(g * 16, 16)]
            l16 = lrankv[pl.ds(g * 16, 16)]
            o16 = plsc.load_gather(offsv, [r16])
            rank = l16 + o16
            loc = r16 - nbase
            m = jnp.logical_and(rank < SAMP,
                                jnp.logical_and(loc >= 0, loc < RN))
            idx = jnp.where(m, loc * SLOT + rank, 0)
            plsc.store_scatter(slabv, [idx], c16 + 1, mask=m)
            return carry

        lax.fori_loop(0, GPW, body, jnp.int32(0))
        pltpu.sync_copy(slabv, part_hbm.at[wid * NRANGE + p])


def _nbr_build(row, col, lrank, offs, zeros_slab):
    k = functools.partial(
        pl.kernel,
        out_type=jax.ShapeDtypeStruct((NW * NRANGE, SLAB), jnp.int32),
        mesh=_sc_mesh(),
        compiler_params=_SC_PARAMS,
        scratch_types=[pltpu.VMEM((EPW,), jnp.int32),
                       pltpu.VMEM((EPW,), jnp.int32),
                       pltpu.VMEM((EPW,), jnp.int32),
                       pltpu.VMEM((N2,), jnp.int32),
                       pltpu.VMEM((SLAB,), jnp.int32)],
    )(_nbr_build_kernel)
    return k(row, col, lrank, offs, zeros_slab)


def _nbr_sum_body(part_ref, out_ref):
    p = part_ref[...]
    s = jnp.sum(p, axis=0, keepdims=True)
    out_ref[...] = jnp.where(s > 0, s - 1, N)


def _nbr_sum(parts2d):
    total = NRANGE * SLAB
    bw = 1280
    grid = (total // bw,)
    return pl.pallas_call(
        _nbr_sum_body,
        out_shape=jax.ShapeDtypeStruct((1, total), jnp.int32),
        grid=grid,
        in_specs=[pl.BlockSpec((NW, bw), lambda i: (0, i))],
        out_specs=pl.BlockSpec((1, bw), lambda i: (0, i)),
    )(parts2d)


def _sum_chunk(buf, accv, chunk):
    def nbody(n, carry):
        row = chunk * CH + n
        for c in range(8):
            sl = pl.ds(c * 16, 16)
            v = buf[10 * n, sl]
            for j in range(1, SLOT):
                v = v + buf[10 * n + j, sl]
            accv[row, sl] = v
        return carry

    lax.fori_loop(0, CH, nbody, jnp.int32(0))


def _pull_pass(tab_hbm, nbv, bufs, accv, sems, nch):
    rows = CH * SLOT
    nbuf = len(bufs)

    def issue(buf, sem, chunk):
        pltpu.make_async_copy(
            tab_hbm.at[nbv.at[pl.ds(chunk * rows, rows)]], buf, sem).start()

    def wait(buf, sem):
        pltpu.make_async_copy(
            tab_hbm.at[nbv.at[pl.ds(0, rows)]], buf, sem).wait()

    for t in range(nbuf):
        issue(bufs[t], sems[t], jnp.int32(t))

    def body(i, carry):
        m = lax.rem(i, nbuf)
        for t in range(nbuf):
            @pl.when(m == t)
            def _():
                wait(bufs[t], sems[t])
                _sum_chunk(bufs[t], accv, i)

                @pl.when(i < nch - nbuf)
                def _():
                    issue(bufs[t], sems[t], i + nbuf)

        return carry

    lax.fori_loop(0, nch, body, jnp.int32(0))


def _pull_worker(tabs, outs, nbrt_hbm, nbv, bufs, accv, sems, nbase, npw):
    pltpu.sync_copy(nbrt_hbm.at[pl.ds(nbase * SLOT, npw * SLOT)],
                    nbv.at[pl.ds(0, npw * SLOT)])
    for tab_hbm, out_hbm in zip(tabs, outs):
        _pull_pass(tab_hbm, nbv, bufs, accv, sems, npw // CH)
        pltpu.sync_copy(accv.at[pl.ds(0, npw)], out_hbm.at[pl.ds(nbase, npw)])


def _pull_split(cid, sid, tabs, outs, nbrt_hbm, nbv, bufs, accv, sems):
    @pl.when(cid == 0)
    def _():
        _pull_worker(tabs, outs, nbrt_hbm, nbv, bufs, accv, sems,
                     sid * NPW_B, NPW_B)

    @pl.when(cid == 1)
    def _():
        _pull_worker(tabs, outs, nbrt_hbm, nbv, bufs, accv, sems,
                     16 * NPW_B + sid * NPW_A, NPW_A)


def _pull_agg_kernel(tab_hbm, nbrt_hbm, agg_hbm, nbv, b0, b1, b2, b3, accv,
                     s0, s1, s2, s3):
    cid = lax.axis_index("c")
    sid = lax.axis_index("s")
    _pull_split(cid, sid, (tab_hbm,), (agg_hbm,), nbrt_hbm, nbv,
                (b0, b1, b2, b3), accv, (s0, s1, s2, s3))


def _pull_agg(tab, nbrt):
    k = functools.partial(
        pl.kernel,
        out_type=jax.ShapeDtypeStruct((N2, 128), jnp.float32),
        mesh=_sc_mesh(),
        compiler_params=_SC_PARAMS,
        scratch_types=[pltpu.VMEM((NPW_B * SLOT,), jnp.int32),
                       pltpu.VMEM((CH * SLOT, 128), jnp.float32),
                       pltpu.VMEM((CH * SLOT, 128), jnp.float32),
                       pltpu.VMEM((CH * SLOT, 128), jnp.float32),
                       pltpu.VMEM((CH * SLOT, 128), jnp.float32),
                       pltpu.VMEM((NPW_B, 128), jnp.float32),
                       pltpu.SemaphoreType.DMA,
                       pltpu.SemaphoreType.DMA,
                       pltpu.SemaphoreType.DMA,
                       pltpu.SemaphoreType.DMA],
    )(_pull_agg_kernel)
    return k(tab, nbrt)


def _pull_agg2_kernel(tl_hbm, tr_hbm, nbrt_hbm, al_hbm, ar_hbm,
                      nbv, b0, b1, b2, b3, accv, s0, s1, s2, s3):
    cid = lax.axis_index("c")
    sid = lax.axis_index("s")
    _pull_split(cid, sid, (tl_hbm, tr_hbm), (al_hbm, ar_hbm), nbrt_hbm, nbv,
                (b0, b1, b2, b3), accv, (s0, s1, s2, s3))


def _pull_agg2(tl, tr, nbrt):
    k = functools.partial(
        pl.kernel,
        out_type=[jax.ShapeDtypeStruct((N2, 128), jnp.float32),
                  jax.ShapeDtypeStruct((N2, 128), jnp.float32)],
        mesh=_sc_mesh(),
        compiler_params=_SC_PARAMS,
        scratch_types=[pltpu.VMEM((NPW_B * SLOT,), jnp.int32),
                       pltpu.VMEM((CH * SLOT, 128), jnp.float32),
                       pltpu.VMEM((CH * SLOT, 128), jnp.float32),
                       pltpu.VMEM((CH * SLOT, 128), jnp.float32),
                       pltpu.VMEM((CH * SLOT, 128), jnp.float32),
                       pltpu.VMEM((NPW_B, 128), jnp.float32),
                       pltpu.SemaphoreType.DMA,
                       pltpu.SemaphoreType.DMA,
                       pltpu.SemaphoreType.DMA,
                       pltpu.SemaphoreType.DMA],
    )(_pull_agg2_kernel)
    return k(tl, tr, nbrt)


def _layer0_body(x_ref, agg_ref, inv_ref, w0_ref, b0_ref, hl_ref, hr_ref):
    i = pl.program_id(0)
    x = x_ref[...]
    aggm = agg_ref[...] * inv_ref[...]
    w0 = w0_ref[...]
    y = lax.dot_general(x, w0[:, :DIN], (((1,), (1,)), ((), ())),
                        precision=lax.Precision.HIGHEST,
                        preferred_element_type=jnp.float32)
    y = y + lax.dot_general(aggm, w0[:, DIN:], (((1,), (1,)), ((), ())),
                            precision=lax.Precision.HIGHEST,
                            preferred_element_type=jnp.float32)
    h = jnp.maximum(y + b0_ref[...], 0.0)
    rows = i * x.shape[0] + lax.broadcasted_iota(jnp.int32, (x.shape[0], 1),
                                                 0)
    h = jnp.where(rows < N, h, 0.0)
    hl_ref[...] = h[:, :128]
    hr_ref[...] = h[:, 128:]


def _layer0(x, agg, inv, w0, b0r):
    br = 1024
    grid = (N2 // br,)
    return pl.pallas_call(
        _layer0_body,
        out_shape=[jax.ShapeDtypeStruct((N2, 128), jnp.float32),
                   jax.ShapeDtypeStruct((N2, 128), jnp.float32)],
        grid=grid,
        in_specs=[pl.BlockSpec((br, DIN), lambda i: (i, 0)),
                  pl.BlockSpec((br, DIN), lambda i: (i, 0)),
                  pl.BlockSpec((br, 1), lambda i: (i, 0)),
                  pl.BlockSpec((DHID, 2 * DIN), lambda i: (0, 0)),
                  pl.BlockSpec((1, DHID), lambda i: (0, 0))],
        out_specs=[pl.BlockSpec((br, 128), lambda i: (i, 0)),
                   pl.BlockSpec((br, 128), lambda i: (i, 0))],
    )(x, agg, inv, w0, b0r)


def _layer1_body(hl_ref, hr_ref, a1l_ref, a1r_ref, inv_ref, w1_ref, b1_ref,
                 out_ref):
    h = jnp.concatenate([hl_ref[...], hr_ref[...]], axis=1)
    inv = inv_ref[...]
    aggm = jnp.concatenate([a1l_ref[...] * inv, a1r_ref[...] * inv], axis=1)
    w1 = w1_ref[...]
    y = lax.dot_general(h, w1[:, :DHID], (((1,), (1,)), ((), ())),
                        precision=lax.Precision.HIGHEST,
                        preferred_element_type=jnp.float32)
    y = y + lax.dot_general(aggm, w1[:, DHID:], (((1,), (1,)), ((), ())),
                            precision=lax.Precision.HIGHEST,
                            preferred_element_type=jnp.float32)
    out_ref[...] = y + b1_ref[...]


def _layer1(hl, hr, a1l, a1r, inv, w1, b1r):
    br = 1024
    grid = (N2 // br,)
    return pl.pallas_call(
        _layer1_body,
        out_shape=jax.ShapeDtypeStruct((N2, DOUT), jnp.float32),
        grid=grid,
        in_specs=[pl.BlockSpec((br, 128), lambda i: (i, 0)),
                  pl.BlockSpec((br, 128), lambda i: (i, 0)),
                  pl.BlockSpec((br, 128), lambda i: (i, 0)),
                  pl.BlockSpec((br, 128), lambda i: (i, 0)),
                  pl.BlockSpec((br, 1), lambda i: (i, 0)),
                  pl.BlockSpec((DOUT, 2 * DHID), lambda i: (0, 0)),
                  pl.BlockSpec((1, DOUT), lambda i: (0, 0))],
        out_specs=pl.BlockSpec((br, DOUT), lambda i: (i, 0)),
    )(hl, hr, a1l, a1r, inv, w1, b1r)


def kernel(x, edge_index, W0, b0, W1, b1):
    row = edge_index[0]
    col = edge_index[1]

    rowt = row.reshape(E // 16, 16).T
    dupt = _dup16(rowt)
    dup = dupt.T.reshape(E)

    hist, lrank = _count(row, dup)

    offs, inv = _offsets(hist)

    zeros_slab = jnp.zeros((SLAB,), jnp.int32)
    parts = _nbr_build(row, col, lrank, offs, zeros_slab)

    nbr_flat = _nbr_sum(parts.reshape(NW, NRANGE * SLAB))
    arr = nbr_flat.reshape(NRANGE, RN, SLOT)
    nbrt_flat = jnp.concatenate([arr[0], arr[1], arr[2]],
                                axis=0)[:N2].reshape(N2 * SLOT)

    x_aug = jnp.concatenate(
        [x, jnp.zeros((N2 - N, DIN), jnp.float32)], axis=0)

    agg0 = _pull_agg(x_aug, nbrt_flat)

    hl, hr = _layer0(x_aug, agg0, inv, W0, b0[None, :])

    a1l, a1r = _pull_agg2(hl, hr, nbrt_flat)

    out = _layer1(hl, hr, a1l, a1r, inv, W1, b1[None, :])
    return out[:N]


# speedup vs baseline: 1.0245x; 1.0245x over previous
import functools

import jax
import jax.numpy as jnp
from jax import lax
from jax.experimental import pallas as pl
from jax.experimental.pallas import tpu as pltpu
from jax.experimental.pallas import tpu_sc as plsc

N = 10000
N2 = 10240
E = 320000
DIN = 128
DHID = 256
DOUT = 128
SAMP = 10
SLOT = 10
NW = 32
EPW = E // NW
GPW = EPW // 16
NPW = N2 // NW
NPW_A = 128
NPW_B = 512
NRANGE = 3
RN = 3456
SLAB = SLOT * RN
CH = 8
NCH = NPW // CH

_SC_PARAMS = pltpu.CompilerParams(needs_layout_passes=False)


def _sc_mesh():
    return plsc.VectorSubcoreMesh(core_axis_name="c", subcore_axis_name="s")


def _dup16_body(rowt_ref, dupt_ref):
    x = rowt_ref[...]
    sub = lax.broadcasted_iota(jnp.int32, x.shape, 0)
    dup = jnp.zeros(x.shape, jnp.int32)
    for k in range(1, 16):
        rk = jnp.concatenate([x[16 - k:, :], x[:16 - k, :]], axis=0)
        eq = jnp.logical_and(x == rk, sub >= k)
        dup = dup + jnp.where(eq, 1, 0)
    dupt_ref[...] = dup


def _dup16(rowt):
    g = rowt.shape[1]
    bw = 2048
    grid = (pl.cdiv(g, bw),)
    return pl.pallas_call(
        _dup16_body,
        out_shape=jax.ShapeDtypeStruct((16, g), jnp.int32),
        grid=grid,
        in_specs=[pl.BlockSpec((16, bw), lambda i: (0, i))],
        out_specs=pl.BlockSpec((16, bw), lambda i: (0, i)),
    )(rowt)


def _count_kernel(row_hbm, dup_hbm, hist_hbm, lrank_hbm,
                  rowsv, dupv, cntv, lrankv):
    cid = lax.axis_index("c")
    sid = lax.axis_index("s")
    wid = cid * 16 + sid
    base = wid * EPW
    pltpu.sync_copy(row_hbm.at[pl.ds(base, EPW)], rowsv)
    pltpu.sync_copy(dup_hbm.at[pl.ds(base, EPW)], dupv)
    zeros16 = jnp.zeros((16,), jnp.int32)

    def zbody(i, carry):
        cntv[pl.ds(i * 16, 16)] = zeros16
        return carry

    lax.fori_loop(0, N2 // 16, zbody, jnp.int32(0))
    ones16 = jnp.ones((16,), jnp.int32)

    def body(g, carry):
        r16 = rowsv[pl.ds(g * 16, 16)]
        d16 = dupv[pl.ds(g * 16, 16)]
        c16 = plsc.load_gather(cntv, [r16])
        lrankv[pl.ds(g * 16, 16)] = c16 + d16
        plsc.addupdate_scatter(cntv, [r16], ones16)
        return carry

    lax.fori_loop(0, GPW, body, jnp.int32(0))
    pltpu.sync_copy(cntv, hist_hbm.at[wid])
    pltpu.sync_copy(lrankv, lrank_hbm.at[pl.ds(base, EPW)])


def _count(row, dup):
    k = functools.partial(
        pl.kernel,
        out_type=[jax.ShapeDtypeStruct((NW, N2), jnp.int32),
                  jax.ShapeDtypeStruct((E,), jnp.int32)],
        mesh=_sc_mesh(),
        compiler_params=_SC_PARAMS,
        scratch_types=[pltpu.VMEM((EPW,), jnp.int32),
                       pltpu.VMEM((EPW,), jnp.int32),
                       pltpu.VMEM((N2,), jnp.int32),
                       pltpu.VMEM((EPW,), jnp.int32)],
    )(_count_kernel)
    return k(row, dup)


def _offsets_body(hist_ref, offs_ref, inv_ref):
    h = hist_ref[...]
    acc = jnp.zeros((1, h.shape[1]), jnp.int32)
    rows = []
    for w in range(NW):
        rows.append(acc)
        acc = acc + h[w:w + 1, :]
    offs_ref[...] = jnp.concatenate(rows, axis=0)
    deg = jnp.minimum(acc, SAMP)
    deg = jnp.maximum(deg, 1).astype(jnp.float32)
    inv = 1.0 / deg
    r = lax.broadcasted_iota(jnp.int32, (128, 128), 0)
    c = lax.broadcasted_iota(jnp.int32, (128, 128), 1)
    ident = (r == c).astype(jnp.float32)
    inv_ref[...] = lax.dot_general(
        ident, inv, (((1,), (1,)), ((), ())),
        precision=lax.Precision.HIGHEST,
        preferred_element_type=jnp.float32)


def _offsets(hist):
    grid = (N2 // 128,)
    return pl.pallas_call(
        _offsets_body,
        out_shape=[jax.ShapeDtypeStruct((NW, N2), jnp.int32),
                   jax.ShapeDtypeStruct((N2, 1), jnp.float32)],
        grid=grid,
        in_specs=[pl.BlockSpec((NW, 128), lambda i: (0, i))],
        out_specs=[pl.BlockSpec((NW, 128), lambda i: (0, i)),
                   pl.BlockSpec((128, 1), lambda i: (i, 0))],
    )(hist)


def _nbr_build_kernel(row_hbm, col_hbm, lrank_hbm, offs_hbm, z_hbm,
                      part_hbm, rowsv, colsv, lrankv, offsv, slabv):
    cid = lax.axis_index("c")
    sid = lax.axis_index("s")
    wid = cid * 16 + sid
    base = wid * EPW
    pltpu.sync_copy(row_hbm.at[pl.ds(base, EPW)], rowsv)
    pltpu.sync_copy(col_hbm.at[pl.ds(base, EPW)], colsv)
    pltpu.sync_copy(lrank_hbm.at[pl.ds(base, EPW)], lrankv)
    pltpu.sync_copy(offs_hbm.at[wid], offsv)

    for p in range(NRANGE):
        pltpu.sync_copy(z_hbm, slabv)
        nbase = p * RN

        def body(g, carry):
            r16 = rowsv[pl.ds(g * 16, 16)]
            c16 = colsv[pl.ds(g * 16, 16)]
            l16 = lrankv[pl.ds(g * 16, 16)]
            o16 = plsc.load_gather(offsv, [r16])
            rank = l16 + o16
            loc = r16 - nbase
            m = jnp.logical_and(rank < SAMP,
                                jnp.logical_and(loc >= 0, loc < RN))
            idx = jnp.where(m, loc * SLOT + rank, 0)
            plsc.store_scatter(slabv, [idx], c16 + 1, mask=m)
            return carry

        lax.fori_loop(0, GPW, body, jnp.int32(0))
        pltpu.sync_copy(slabv, part_hbm.at[wid * NRANGE + p])


def _nbr_build(row, col, lrank, offs, zeros_slab):
    k = functools.partial(
        pl.kernel,
        out_type=jax.ShapeDtypeStruct((NW * NRANGE, SLAB), jnp.int32),
        mesh=_sc_mesh(),
        compiler_params=_SC_PARAMS,
        scratch_types=[pltpu.VMEM((EPW,), jnp.int32),
                       pltpu.VMEM((EPW,), jnp.int32),
                       pltpu.VMEM((EPW,), jnp.int32),
                       pltpu.VMEM((N2,), jnp.int32),
                       pltpu.VMEM((SLAB,), jnp.int32)],
    )(_nbr_build_kernel)
    return k(row, col, lrank, offs, zeros_slab)


def _nbr_sum_body(part_ref, out_ref):
    p = part_ref[...]
    s = jnp.sum(p, axis=0, keepdims=True)
    out_ref[...] = jnp.where(s > 0, s - 1, N)


def _nbr_sum(parts2d):
    total = NRANGE * SLAB
    bw = 1280
    grid = (total // bw,)
    return pl.pallas_call(
        _nbr_sum_body,
        out_shape=jax.ShapeDtypeStruct((1, total), jnp.int32),
        grid=grid,
        in_specs=[pl.BlockSpec((NW, bw), lambda i: (0, i))],
        out_specs=pl.BlockSpec((1, bw), lambda i: (0, i)),
    )(parts2d)


def _sum_chunk(buf, accv, chunk):
    def nbody(n, carry):
        row = chunk * CH + n
        for c in range(8):
            sl = pl.ds(c * 16, 16)
            v = buf[10 * n, sl]
            for j in range(1, SLOT):
                v = v + buf[10 * n + j, sl]
            accv[row, sl] = v
        return carry

    lax.fori_loop(0, CH, nbody, jnp.int32(0))


def _pull_pass(tab_hbm, nbv, bufs, accv, sems, nch):
    rows = CH * SLOT
    nbuf = len(bufs)

    def issue(buf, sem, chunk):
        pltpu.make_async_copy(
            tab_hbm.at[nbv.at[pl.ds(chunk * rows, rows)]], buf, sem).start()

    def wait(buf, sem):
        pltpu.make_async_copy(
            tab_hbm.at[nbv.at[pl.ds(0, rows)]], buf, sem).wait()

    for t in range(nbuf):
        issue(bufs[t], sems[t], jnp.int32(t))

    def body(i, carry):
        m = lax.rem(i, nbuf)
        for t in range(nbuf):
            @pl.when(m == t)
            def _():
                wait(bufs[t], sems[t])
                _sum_chunk(bufs[t], accv, i)

                @pl.when(i < nch - nbuf)
                def _():
                    issue(bufs[t], sems[t], i + nbuf)

        return carry

    lax.fori_loop(0, nch, body, jnp.int32(0))


def _pull_worker(tabs, outs, nbrt_hbm, nbv, bufs, accv, sems, nbase, npw):
    pltpu.sync_copy(nbrt_hbm.at[pl.ds(nbase * SLOT, npw * SLOT)],
                    nbv.at[pl.ds(0, npw * SLOT)])
    for tab_hbm, out_hbm in zip(tabs, outs):
        _pull_pass(tab_hbm, nbv, bufs, accv, sems, npw // CH)
        pltpu.sync_copy(accv.at[pl.ds(0, npw)], out_hbm.at[pl.ds(nbase, npw)])


def _pull_split(cid, sid, tabs, outs, nbrt_hbm, nbv, bufs, accv, sems):
    @pl.when(cid == 0)
    def _():
        _pull_worker(tabs, outs, nbrt_hbm, nbv, bufs, accv, sems,
                     sid * NPW_B, NPW_B)

    @pl.when(cid == 1)
    def _():
        _pull_worker(tabs, outs, nbrt_hbm, nbv, bufs, accv, sems,
                     16 * NPW_B + sid * NPW_A, NPW_A)


def _pull_agg_kernel(tab_hbm, nbrt_hbm, agg_hbm, nbv, b0, b1, b2, b3, accv,
                     s0, s1, s2, s3):
    cid = lax.axis_index("c")
    sid = lax.axis_index("s")
    _pull_split(cid, sid, (tab_hbm,), (agg_hbm,), nbrt_hbm, nbv,
                (b0, b1, b2, b3), accv, (s0, s1, s2, s3))


def _pull_agg(tab, nbrt):
    k = functools.partial(
        pl.kernel,
        out_type=jax.ShapeDtypeStruct((N2, 128), jnp.float32),
        mesh=_sc_mesh(),
        compiler_params=_SC_PARAMS,
        scratch_types=[pltpu.VMEM((NPW_B * SLOT,), jnp.int32),
                       pltpu.VMEM((CH * SLOT, 128), jnp.float32),
                       pltpu.VMEM((CH * SLOT, 128), jnp.float32),
                       pltpu.VMEM((CH * SLOT, 128), jnp.float32),
                       pltpu.VMEM((CH * SLOT, 128), jnp.float32),
                       pltpu.VMEM((NPW_B, 128), jnp.float32),
                       pltpu.SemaphoreType.DMA,
                       pltpu.SemaphoreType.DMA,
                       pltpu.SemaphoreType.DMA,
                       pltpu.SemaphoreType.DMA],
    )(_pull_agg_kernel)
    return k(tab, nbrt)


def _pull_agg2_kernel(tl_hbm, tr_hbm, nbrt_hbm, al_hbm, ar_hbm,
                      nbv, b0, b1, b2, b3, accv, s0, s1, s2, s3):
    cid = lax.axis_index("c")
    sid = lax.axis_index("s")
    _pull_split(cid, sid, (tl_hbm, tr_hbm), (al_hbm, ar_hbm), nbrt_hbm, nbv,
                (b0, b1, b2, b3), accv, (s0, s1, s2, s3))


def _pull_agg2(tl, tr, nbrt):
    k = functools.partial(
        pl.kernel,
        out_type=[jax.ShapeDtypeStruct((N2, 128), jnp.float32),
                  jax.ShapeDtypeStruct((N2, 128), jnp.float32)],
        mesh=_sc_mesh(),
        compiler_params=_SC_PARAMS,
        scratch_types=[pltpu.VMEM((NPW_B * SLOT,), jnp.int32),
                       pltpu.VMEM((CH * SLOT, 128), jnp.float32),
                       pltpu.VMEM((CH * SLOT, 128), jnp.float32),
                       pltpu.VMEM((CH * SLOT, 128), jnp.float32),
                       pltpu.VMEM((CH * SLOT, 128), jnp.float32),
                       pltpu.VMEM((NPW_B, 128), jnp.float32),
                       pltpu.SemaphoreType.DMA,
                       pltpu.SemaphoreType.DMA,
                       pltpu.SemaphoreType.DMA,
                       pltpu.SemaphoreType.DMA],
    )(_pull_agg2_kernel)
    return k(tl, tr, nbrt)


def _layer0_body(x_ref, agg_ref, inv_ref, w0_ref, b0_ref, hl_ref, hr_ref):
    i = pl.program_id(0)
    x = x_ref[...]
    aggm = agg_ref[...] * inv_ref[...]
    w0 = w0_ref[...]
    y = lax.dot_general(x, w0[:, :DIN], (((1,), (1,)), ((), ())),
                        precision=lax.Precision.HIGHEST,
                        preferred_element_type=jnp.float32)
    y = y + lax.dot_general(aggm, w0[:, DIN:], (((1,), (1,)), ((), ())),
                            precision=lax.Precision.HIGHEST,
                            preferred_element_type=jnp.float32)
    h = jnp.maximum(y + b0_ref[...], 0.0)
    rows = i * x.shape[0] + lax.broadcasted_iota(jnp.int32, (x.shape[0], 1),
                                                 0)
    h = jnp.where(rows < N, h, 0.0)
    hl_ref[...] = h[:, :128]
    hr_ref[...] = h[:, 128:]


def _layer0(x, agg, inv, w0, b0r):
    br = 1024
    grid = (N2 // br,)
    return pl.pallas_call(
        _layer0_body,
        out_shape=[jax.ShapeDtypeStruct((N2, 128), jnp.float32),
                   jax.ShapeDtypeStruct((N2, 128), jnp.float32)],
        grid=grid,
        in_specs=[pl.BlockSpec((br, DIN), lambda i: (i, 0)),
                  pl.BlockSpec((br, DIN), lambda i: (i, 0)),
                  pl.BlockSpec((br, 1), lambda i: (i, 0)),
                  pl.BlockSpec((DHID, 2 * DIN), lambda i: (0, 0)),
                  pl.BlockSpec((1, DHID), lambda i: (0, 0))],
        out_specs=[pl.BlockSpec((br, 128), lambda i: (i, 0)),
                   pl.BlockSpec((br, 128), lambda i: (i, 0))],
    )(x, agg, inv, w0, b0r)


def _layer1_body(hl_ref, hr_ref, a1l_ref, a1r_ref, inv_ref, w1_ref, b1_ref,
                 out_ref):
    h = jnp.concatenate([hl_ref[...], hr_ref[...]], axis=1)
    inv = inv_ref[...]
    aggm = jnp.concatenate([a1l_ref[...] * inv, a1r_ref[...] * inv], axis=1)
    w1 = w1_ref[...]
    y = lax.dot_general(h, w1[:, :DHID], (((1,), (1,)), ((), ())),
                        precision=lax.Precision.HIGHEST,
                        preferred_element_type=jnp.float32)
    y = y + lax.dot_general(aggm, w1[:, DHID:], (((1,), (1,)), ((), ())),
                            precision=lax.Precision.HIGHEST,
                            preferred_element_type=jnp.float32)
    out_ref[...] = y + b1_ref[...]


def _layer1(hl, hr, a1l, a1r, inv, w1, b1r):
    br = 1024
    grid = (N2 // br,)
    return pl.pallas_call(
        _layer1_body,
        out_shape=jax.ShapeDtypeStruct((N2, DOUT), jnp.float32),
        grid=grid,
        in_specs=[pl.BlockSpec((br, 128), lambda i: (i, 0)),
                  pl.BlockSpec((br, 128), lambda i: (i, 0)),
                  pl.BlockSpec((br, 128), lambda i: (i, 0)),
                  pl.BlockSpec((br, 128), lambda i: (i, 0)),
                  pl.BlockSpec((br, 1), lambda i: (i, 0)),
                  pl.BlockSpec((DOUT, 2 * DHID), lambda i: (0, 0)),
                  pl.BlockSpec((1, DOUT), lambda i: (0, 0))],
        out_specs=pl.BlockSpec((br, DOUT), lambda i: (i, 0)),
    )(hl, hr, a1l, a1r, inv, w1, b1r)


def kernel(x, edge_index, W0, b0, W1, b1):
    row = edge_index[0]
    col = edge_index[1]

    rowt = row.reshape(E // 16, 16).T
    dupt = _dup16(rowt)
    dup = dupt.T.reshape(E)

    hist, lrank = _count(row, dup)

    offs, inv = _offsets(hist)

    zeros_slab = jnp.zeros((SLAB,), jnp.int32)
    parts = _nbr_build(row, col, lrank, offs, zeros_slab)

    nbr_flat = _nbr_sum(parts.reshape(NW, NRANGE * SLAB))
    arr = nbr_flat.reshape(NRANGE, RN, SLOT)
    nbrt_flat = jnp.concatenate([arr[0], arr[1], arr[2]],
                                axis=0)[:N2].reshape(N2 * SLOT)

    x_aug = jnp.concatenate(
        [x, jnp.zeros((N2 - N, DIN), jnp.float32)], axis=0)

    agg0 = _pull_agg(x_aug, nbrt_flat)

    hl, hr = _layer0(x_aug, agg0, inv, W0, b0[None, :])

    a1l, a1r = _pull_agg2(hl, hr, nbrt_flat)

    out = _layer1(hl, hr, a1l, a1r, inv, W1, b1[None, :])
    return out[:N]
